# Initial kernel scaffold; baseline (speedup 1.0000x reference)
import jax
import jax.numpy as jnp
from jax.experimental import pallas as pl


def kernel(x, edge_index):
    raise NotImplementedError("write your pallas kernel here")


# speedup vs baseline: 2.8055x; 2.8055x over previous
import functools

import jax
import jax.numpy as jnp
from jax import lax
from jax.experimental import pallas as pl
from jax.experimental.pallas import tpu as pltpu
from jax.experimental.pallas import tpu_sc as plsc

_D = 128
_E = 320000
_NC, _NS = 2, 16
_NW = _NC * _NS
_C = 80
_EPW = _E // _NW
_NCHUNK = _EPW // _C

_mesh = plsc.VectorSubcoreMesh(core_axis_name="c", subcore_axis_name="s")


@functools.partial(
    pl.kernel,
    mesh=_mesh,
    out_type=jax.ShapeDtypeStruct((_E, _D), jnp.float32),
    scratch_types=[
        pltpu.VMEM((_C,), jnp.int32),
        pltpu.VMEM((_C, _D), jnp.float32),
        pltpu.SemaphoreType.DMA,
    ],
)
def _gather_kernel(src_hbm, x_hbm, out_hbm, idx_v, rows_v, sem):
    wid = lax.axis_index("s") * _NC + lax.axis_index("c")
    base_w = wid * _EPW

    def chunk(j, carry):
        off = base_w + j * _C
        pltpu.sync_copy(src_hbm.at[pl.ds(off, _C)], idx_v)
        pltpu.async_copy(x_hbm.at[idx_v], rows_v, sem).wait()
        pltpu.sync_copy(rows_v, out_hbm.at[pl.ds(off, _C)])
        return carry

    lax.fori_loop(0, _NCHUNK, chunk, 0)


def kernel(x, edge_index):
    src = edge_index[0].astype(jnp.int32)
    messages = _gather_kernel(src, x)
    return (x, edge_index, messages)


# speedup vs baseline: 5.4219x; 1.9326x over previous
import functools

import jax
import jax.numpy as jnp
from jax import lax
from jax.experimental import pallas as pl
from jax.experimental.pallas import tpu as pltpu
from jax.experimental.pallas import tpu_sc as plsc

_D = 128
_E = 320000
_NC, _NS = 2, 16
_NW = _NC * _NS
_C = 80
_NBUF = 5
_EPW = _E // _NW
_NCHUNK = _EPW // _C
_NGROUP = _NCHUNK // _NBUF

_mesh = plsc.VectorSubcoreMesh(core_axis_name="c", subcore_axis_name="s")


@functools.partial(
    pl.kernel,
    mesh=_mesh,
    out_type=jax.ShapeDtypeStruct((_E, _D), jnp.float32),
    scratch_types=(
        [pltpu.VMEM((_EPW,), jnp.int32)]
        + [pltpu.VMEM((_C, _D), jnp.float32) for _ in range(_NBUF)]
        + [pltpu.SemaphoreType.DMA for _ in range(_NBUF)]
        + [pltpu.SemaphoreType.DMA for _ in range(_NBUF)]
    ),
)
def _gather_kernel(src_hbm, x_hbm, out_hbm, *scr):
    idx_all = scr[0]
    rows = scr[1:1 + _NBUF]
    gsem = scr[1 + _NBUF:1 + 2 * _NBUF]
    ssem = scr[1 + 2 * _NBUF:1 + 3 * _NBUF]

    wid = lax.axis_index("s") * _NC + lax.axis_index("c")
    base_w = wid * _EPW

    pltpu.sync_copy(src_hbm.at[pl.ds(base_w, _EPW)], idx_all)

    def group(g, carry):
        base_g = base_w + g * (_NBUF * _C)
        loc_g = g * (_NBUF * _C)
        for b in range(_NBUF):
            idx_c = idx_all.at[pl.ds(loc_g + b * _C, _C)]

            @pl.when(g > 0)
            def _wait_prev_store(b=b):
                pltpu.make_async_copy(
                    rows[b], out_hbm.at[pl.ds(base_w, _C)], ssem[b]
                ).wait()

            pltpu.async_copy(x_hbm.at[idx_c], rows[b], gsem[b])
        for b in range(_NBUF):
            idx_c = idx_all.at[pl.ds(loc_g + b * _C, _C)]
            pltpu.make_async_copy(x_hbm.at[idx_c], rows[b], gsem[b]).wait()
            pltpu.async_copy(
                rows[b], out_hbm.at[pl.ds(base_g + b * _C, _C)], ssem[b]
            )
        return carry

    lax.fori_loop(0, _NGROUP, group, 0)

    for b in range(_NBUF):
        pltpu.make_async_copy(
            rows[b], out_hbm.at[pl.ds(base_w, _C)], ssem[b]
        ).wait()


def kernel(x, edge_index):
    src = edge_index[0].astype(jnp.int32)
    messages = _gather_kernel(src, x)
    return (x, edge_index, messages)


# speedup vs baseline: 7.9042x; 1.4578x over previous
import functools

import jax
import jax.numpy as jnp
from jax import lax
from jax.experimental import pallas as pl
from jax.experimental.pallas import tpu as pltpu
from jax.experimental.pallas import tpu_sc as plsc

_N = 10000
_D = 128
_E = 320000
_NC, _NS = 2, 16
_NW = _NC * _NS
_C = 40
_NBUF = 5
_EPW = _E // _NW
_NCHUNK = _EPW // _C
_NGROUP = _NCHUNK // _NBUF

_mesh = plsc.VectorSubcoreMesh(core_axis_name="c", subcore_axis_name="s")


@functools.partial(
    pl.kernel,
    mesh=_mesh,
    out_type=jax.ShapeDtypeStruct((_E, _D), jnp.float32),
    scratch_types=(
        [pltpu.VMEM_SHARED((_N, _D), jnp.float32)]
        + [pltpu.VMEM((_EPW,), jnp.int32)]
        + [pltpu.VMEM((_C, _D), jnp.float32) for _ in range(_NBUF)]
        + [pltpu.SemaphoreType.DMA for _ in range(_NBUF)]
        + [pltpu.SemaphoreType.DMA for _ in range(_NBUF)]
    ),
)
def _gather_kernel(src_hbm, x_hbm, out_hbm, *scr):
    x_sp = scr[0]
    idx_all = scr[1]
    rows = scr[2:2 + _NBUF]
    gsem = scr[2 + _NBUF:2 + 2 * _NBUF]
    ssem = scr[2 + 2 * _NBUF:2 + 3 * _NBUF]

    cid = lax.axis_index("c")
    sid = lax.axis_index("s")
    wid = sid * _NC + cid
    base_w = wid * _EPW

    pltpu.sync_copy(src_hbm.at[pl.ds(base_w, _EPW)], idx_all)

    @pl.when(sid == 0)
    def _stage_table():
        pltpu.sync_copy(x_hbm, x_sp)

    plsc.subcore_barrier()

    def group(g, carry):
        base_g = base_w + g * (_NBUF * _C)
        loc_g = g * (_NBUF * _C)
        for b in range(_NBUF):
            idx_c = idx_all.at[pl.ds(loc_g + b * _C, _C)]

            @pl.when(g > 0)
            def _wait_prev_store(b=b):
                pltpu.make_async_copy(
                    rows[b], out_hbm.at[pl.ds(base_w, _C)], ssem[b]
                ).wait()

            pltpu.async_copy(x_sp.at[idx_c], rows[b], gsem[b])
        for b in range(_NBUF):
            idx_c = idx_all.at[pl.ds(loc_g + b * _C, _C)]
            pltpu.make_async_copy(x_sp.at[idx_c], rows[b], gsem[b]).wait()
            pltpu.async_copy(
                rows[b], out_hbm.at[pl.ds(base_g + b * _C, _C)], ssem[b]
            )
        return carry

    lax.fori_loop(0, _NGROUP, group, 0)

    for b in range(_NBUF):
        pltpu.make_async_copy(
            rows[b], out_hbm.at[pl.ds(base_w, _C)], ssem[b]
        ).wait()


def kernel(x, edge_index):
    src = edge_index[0].astype(jnp.int32)
    messages = _gather_kernel(src, x)
    return (x, edge_index, messages)


# speedup vs baseline: 7.9063x; 1.0003x over previous
import functools

import jax
import jax.numpy as jnp
from jax import lax
from jax.experimental import pallas as pl
from jax.experimental.pallas import tpu as pltpu
from jax.experimental.pallas import tpu_sc as plsc

_N = 10000
_D = 128
_E = 320000
_NC, _NS = 2, 16
_NW = _NC * _NS
_C = 40
_NBUF = 5
_EPW = _E // _NW
_NCHUNK = _EPW // _C
_NGROUP = _NCHUNK // _NBUF

_mesh = plsc.VectorSubcoreMesh(core_axis_name="c", subcore_axis_name="s")


@functools.partial(
    pl.kernel,
    mesh=_mesh,
    out_type=jax.ShapeDtypeStruct((_E, _D), jnp.float32),
    scratch_types=(
        [pltpu.VMEM_SHARED((_N, _D), jnp.float32)]
        + [pltpu.VMEM((_EPW,), jnp.int32)]
        + [pltpu.VMEM((_C, _D), jnp.float32) for _ in range(_NBUF)]
        + [pltpu.SemaphoreType.DMA for _ in range(_NBUF)]
        + [pltpu.SemaphoreType.DMA for _ in range(_NBUF)]
    ),
)
def _gather_kernel(src_hbm, x_hbm, out_hbm, *scr):
    x_sp = scr[0]
    idx_all = scr[1]
    rows = scr[2:2 + _NBUF]
    gsem = scr[2 + _NBUF:2 + 2 * _NBUF]
    ssem = scr[2 + 2 * _NBUF:2 + 3 * _NBUF]

    cid = lax.axis_index("c")
    sid = lax.axis_index("s")
    wid = sid * _NC + cid
    base_w = wid * _EPW

    pltpu.sync_copy(src_hbm.at[pl.ds(base_w, _EPW)], idx_all)

    @pl.when(sid < _NS - 1)
    def _stage_main():
        pltpu.sync_copy(
            x_hbm.at[pl.ds(sid * 624, 624)], x_sp.at[pl.ds(sid * 624, 624)]
        )

    @pl.when(sid == _NS - 1)
    def _stage_tail():
        pltpu.sync_copy(
            x_hbm.at[pl.ds((_NS - 1) * 624, _N - (_NS - 1) * 624)],
            x_sp.at[pl.ds((_NS - 1) * 624, _N - (_NS - 1) * 624)],
        )

    plsc.subcore_barrier()

    def group(g, carry):
        base_g = base_w + g * (_NBUF * _C)
        loc_g = g * (_NBUF * _C)
        for b in range(_NBUF):
            idx_c = idx_all.at[pl.ds(loc_g + b * _C, _C)]

            @pl.when(g > 0)
            def _wait_prev_store(b=b):
                pltpu.make_async_copy(
                    rows[b], out_hbm.at[pl.ds(base_w, _C)], ssem[b]
                ).wait()

            pltpu.async_copy(x_sp.at[idx_c], rows[b], gsem[b])
        for b in range(_NBUF):
            idx_c = idx_all.at[pl.ds(loc_g + b * _C, _C)]
            pltpu.make_async_copy(x_sp.at[idx_c], rows[b], gsem[b]).wait()
            pltpu.async_copy(
                rows[b], out_hbm.at[pl.ds(base_g + b * _C, _C)], ssem[b]
            )
        return carry

    lax.fori_loop(0, _NGROUP, group, 0)

    for b in range(_NBUF):
        pltpu.make_async_copy(
            rows[b], out_hbm.at[pl.ds(base_w, _C)], ssem[b]
        ).wait()


def kernel(x, edge_index):
    src = edge_index[0].astype(jnp.int32)
    messages = _gather_kernel(src, x)
    return (x, edge_index, messages)


# speedup vs baseline: 7.9300x; 1.0030x over previous
import functools

import jax
import jax.numpy as jnp
from jax import lax
from jax.experimental import pallas as pl
from jax.experimental.pallas import tpu as pltpu
from jax.experimental.pallas import tpu_sc as plsc

_N = 10000
_D = 128
_E = 320000
_NC, _NS = 2, 16
_NW = _NC * _NS
_C = 40
_NBUF = 5
_EPW = _E // _NW
_NCHUNK = _EPW // _C
_NGROUP = _NCHUNK // _NBUF

_mesh = plsc.VectorSubcoreMesh(core_axis_name="c", subcore_axis_name="s")


@functools.partial(
    pl.kernel,
    mesh=_mesh,
    out_type=jax.ShapeDtypeStruct((_E, _D), jnp.float32),
    scratch_types=(
        [pltpu.VMEM_SHARED((_N, _D), jnp.float32)]
        + [pltpu.VMEM((_EPW,), jnp.int32)]
        + [pltpu.VMEM((_C, _D), jnp.float32) for _ in range(_NBUF)]
        + [pltpu.SemaphoreType.DMA for _ in range(_NBUF)]
        + [pltpu.SemaphoreType.DMA for _ in range(_NBUF)]
    ),
)
def _gather_kernel(src_hbm, x_hbm, out_hbm, *scr):
    x_sp = scr[0]
    idx_all = scr[1]
    rows = scr[2:2 + _NBUF]
    gsem = scr[2 + _NBUF:2 + 2 * _NBUF]
    ssem = scr[2 + 2 * _NBUF:2 + 3 * _NBUF]

    cid = lax.axis_index("c")
    sid = lax.axis_index("s")
    wid = sid * _NC + cid
    base_w = wid * _EPW

    pltpu.sync_copy(src_hbm.at[pl.ds(base_w, _EPW)], idx_all)

    for b in range(_NBUF):
        pltpu.async_copy(
            x_hbm.at[idx_all.at[pl.ds(b * _C, _C)]], rows[b], gsem[b]
        )

    @pl.when(sid < _NS - 1)
    def _stage_main():
        pltpu.sync_copy(
            x_hbm.at[pl.ds(sid * 624, 624)], x_sp.at[pl.ds(sid * 624, 624)]
        )

    @pl.when(sid == _NS - 1)
    def _stage_tail():
        pltpu.sync_copy(
            x_hbm.at[pl.ds((_NS - 1) * 624, _N - (_NS - 1) * 624)],
            x_sp.at[pl.ds((_NS - 1) * 624, _N - (_NS - 1) * 624)],
        )

    plsc.subcore_barrier()

    for b in range(_NBUF):
        pltpu.make_async_copy(
            x_hbm.at[idx_all.at[pl.ds(b * _C, _C)]], rows[b], gsem[b]
        ).wait()
        pltpu.async_copy(
            rows[b], out_hbm.at[pl.ds(base_w + b * _C, _C)], ssem[b]
        )

    def group(g, carry):
        base_g = base_w + g * (_NBUF * _C)
        loc_g = g * (_NBUF * _C)
        for b in range(_NBUF):
            idx_c = idx_all.at[pl.ds(loc_g + b * _C, _C)]
            pltpu.make_async_copy(
                rows[b], out_hbm.at[pl.ds(base_w, _C)], ssem[b]
            ).wait()
            pltpu.async_copy(x_sp.at[idx_c], rows[b], gsem[b])
        for b in range(_NBUF):
            idx_c = idx_all.at[pl.ds(loc_g + b * _C, _C)]
            pltpu.make_async_copy(x_sp.at[idx_c], rows[b], gsem[b]).wait()
            pltpu.async_copy(
                rows[b], out_hbm.at[pl.ds(base_g + b * _C, _C)], ssem[b]
            )
        return carry

    lax.fori_loop(1, _NGROUP, group, 0)

    for b in range(_NBUF):
        pltpu.make_async_copy(
            rows[b], out_hbm.at[pl.ds(base_w, _C)], ssem[b]
        ).wait()


def kernel(x, edge_index):
    src = edge_index[0].astype(jnp.int32)
    messages = _gather_kernel(src, x)
    return (x, edge_index, messages)


# speedup vs baseline: 8.7776x; 1.1069x over previous
import functools

import jax
import jax.numpy as jnp
from jax import lax
from jax.experimental import pallas as pl
from jax.experimental.pallas import tpu as pltpu
from jax.experimental.pallas import tpu_sc as plsc

_N = 10000
_D = 128
_E = 320000
_NC, _NS = 2, 16
_NW = _NC * _NS
_C = 40
_NBUF = 5
_EPW = _E // _NW
_NCHUNK = _EPW // _C
_NGROUP = _NCHUNK // _NBUF

_mesh = plsc.VectorSubcoreMesh(core_axis_name="c", subcore_axis_name="s")


@functools.partial(
    pl.kernel,
    mesh=_mesh,
    out_type=jax.ShapeDtypeStruct((_E, _D), jnp.float32),
    scratch_types=(
        [pltpu.VMEM_SHARED((_N, _D), jnp.float32)]
        + [pltpu.VMEM((_EPW,), jnp.int32)]
        + [pltpu.VMEM((_C, _D), jnp.float32) for _ in range(_NBUF)]
        + [pltpu.SemaphoreType.DMA for _ in range(_NBUF)]
        + [pltpu.SemaphoreType.DMA for _ in range(_NBUF)]
    ),
)
def _gather_kernel(src_hbm, x_hbm, out_hbm, *scr):
    x_sp = scr[0]
    idx_all = scr[1]
    rows = scr[2:2 + _NBUF]
    gsem = scr[2 + _NBUF:2 + 2 * _NBUF]
    ssem = scr[2 + 2 * _NBUF:2 + 3 * _NBUF]

    cid = lax.axis_index("c")
    sid = lax.axis_index("s")
    wid = sid * _NC + cid
    base_w = wid * _EPW

    pltpu.sync_copy(src_hbm.at[pl.ds(base_w, _EPW)], idx_all)

    for b in range(_NBUF):
        pltpu.async_copy(
            x_hbm.at[idx_all.at[pl.ds(b * _C, _C)]], rows[b], gsem[b]
        )

    @pl.when(sid < _NS - 1)
    def _stage_main():
        pltpu.sync_copy(
            x_hbm.at[pl.ds(sid * 624, 624)], x_sp.at[pl.ds(sid * 624, 624)]
        )

    @pl.when(sid == _NS - 1)
    def _stage_tail():
        pltpu.sync_copy(
            x_hbm.at[pl.ds((_NS - 1) * 624, _N - (_NS - 1) * 624)],
            x_sp.at[pl.ds((_NS - 1) * 624, _N - (_NS - 1) * 624)],
        )

    plsc.subcore_barrier()

    for b in range(_NBUF):
        pltpu.make_async_copy(
            x_hbm.at[idx_all.at[pl.ds(b * _C, _C)]], rows[b], gsem[b]
        ).wait()
        pltpu.async_copy(
            rows[b], out_hbm.at[pl.ds(base_w + b * _C, _C)], ssem[b]
        )

    def group(g, carry):
        base_g = base_w + g * (_NBUF * _C)
        loc_g = g * (_NBUF * _C)
        for b in range(_NBUF):
            idx_c = idx_all.at[pl.ds(loc_g + b * _C, _C)]
            pltpu.make_async_copy(
                rows[b], out_hbm.at[pl.ds(base_w, _C)], ssem[b]
            ).wait()
            pltpu.async_copy(x_sp.at[idx_c], rows[b], gsem[b])
        for b in range(_NBUF):
            idx_c = idx_all.at[pl.ds(loc_g + b * _C, _C)]
            pltpu.make_async_copy(x_sp.at[idx_c], rows[b], gsem[b]).wait()
            pltpu.async_copy(
                rows[b], out_hbm.at[pl.ds(base_g + b * _C, _C)], ssem[b]
            )
        return carry

    lax.fori_loop(1, _NGROUP, group, 0)

    for b in range(_NBUF):
        pltpu.make_async_copy(
            rows[b], out_hbm.at[pl.ds(base_w, _C)], ssem[b]
        ).wait()


def kernel(x, edge_index):
    src = edge_index.astype(jnp.int32).reshape(-1)
    messages = _gather_kernel(src, x)
    return (x, edge_index, messages)
